# baseline (speedup 1.0000x reference)
import functools

import jax
import jax.numpy as jnp
from jax import lax
from jax.experimental import pallas as pl
from jax.experimental.pallas import tpu as pltpu
from jax.experimental.pallas import tpu_sc as plsc

_NC = 2
_NS = 16
_NW = _NC * _NS

_SUB = 128
_N_SUB = 8
_CHUNK = _SUB * _N_SUB


@functools.lru_cache(maxsize=None)
def _make_gather(B, V, D):
    b_per_w = B // _NW
    n_chunks = b_per_w // _CHUNK
    assert b_per_w * _NW == B and n_chunks * _CHUNK == b_per_w

    mesh = plsc.VectorSubcoreMesh(core_axis_name="c", subcore_axis_name="s")

    @functools.partial(
        pl.kernel,
        out_type=jax.ShapeDtypeStruct((B, D), jnp.float32),
        mesh=mesh,
        scratch_types=[
            pltpu.VMEM((_N_SUB, _SUB), jnp.int32),
            pltpu.VMEM((_CHUNK, D), jnp.float32),
            pltpu.SemaphoreType.DMA,
        ],
        compiler_params=pltpu.CompilerParams(use_tc_tiling_on_sc=False),
    )
    def k(table_hbm, idx_hbm, out_hbm, idx_v, rows_v, sem):
        wid = lax.axis_index("s") * _NC + lax.axis_index("c")
        base = wid * b_per_w

        def body(i, carry):
            off = pl.multiple_of(base + i * _CHUNK, _CHUNK)
            row = pl.multiple_of(off // _SUB, _N_SUB)
            pltpu.sync_copy(idx_hbm.at[pl.ds(row, _N_SUB)], idx_v)
            copies = [
                pltpu.async_copy(
                    table_hbm.at[idx_v.at[j]],
                    rows_v.at[pl.ds(j * _SUB, _SUB)],
                    sem,
                )
                for j in range(_N_SUB)
            ]
            for c in copies:
                c.wait()
            pltpu.sync_copy(rows_v, out_hbm.at[pl.ds(off, _CHUNK)])
            return carry

        lax.fori_loop(0, n_chunks, body, 0)

    return k


def kernel(words, weight):
    B = words.size
    V, D = weight.shape
    idx = words.reshape(B // _SUB, _SUB).astype(jnp.int32)
    out = _make_gather(B, V, D)(weight, idx)
    return out.reshape(*words.shape, D)


# speedup vs baseline: 1.0023x; 1.0023x over previous
import functools

import jax
import jax.numpy as jnp
from jax import lax
from jax.experimental import pallas as pl
from jax.experimental.pallas import tpu as pltpu
from jax.experimental.pallas import tpu_sc as plsc

_NC = 2
_NS = 16
_NW = _NC * _NS

_SUB = 128
_N_SUB = 4
_CHUNK = _SUB * _N_SUB


@functools.lru_cache(maxsize=None)
def _make_gather(B, V, D):
    b_per_w = B // _NW
    n_chunks = b_per_w // _CHUNK
    n_pairs = n_chunks // 2
    assert b_per_w * _NW == B and n_pairs * 2 * _CHUNK == b_per_w

    mesh = plsc.VectorSubcoreMesh(core_axis_name="c", subcore_axis_name="s")

    @functools.partial(
        pl.kernel,
        out_type=jax.ShapeDtypeStruct((B, D), jnp.float32),
        mesh=mesh,
        scratch_types=[
            pltpu.VMEM((2, _N_SUB, _SUB), jnp.int32),
            pltpu.VMEM((2, _CHUNK, D), jnp.float32),
            pltpu.SemaphoreType.DMA,
            pltpu.SemaphoreType.DMA,
        ],
        compiler_params=pltpu.CompilerParams(use_tc_tiling_on_sc=False),
    )
    def k(table_hbm, idx_hbm, out_hbm, idx_v, rows_v, gsem, ssem):
        wid = lax.axis_index("s") * _NC + lax.axis_index("c")
        base = wid * b_per_w

        def out_slice(i):
            return out_hbm.at[pl.ds(pl.multiple_of(base + i * _CHUNK, _CHUNK), _CHUNK)]

        def idx_load(i, b):
            row = pl.multiple_of((base + i * _CHUNK) // _SUB, _N_SUB)
            pltpu.sync_copy(idx_hbm.at[pl.ds(row, _N_SUB)], idx_v.at[b])

        def gather_start(b):
            for j in range(_N_SUB):
                pltpu.async_copy(
                    table_hbm.at[idx_v.at[b, j]],
                    rows_v.at[b, pl.ds(j * _SUB, _SUB)],
                    gsem,
                )

        def gather_wait(b):
            pltpu.make_async_copy(out_slice(0), rows_v.at[b], gsem).wait()

        def store_start(i, b):
            pltpu.async_copy(rows_v.at[b], out_slice(i), ssem)

        def store_wait(b):
            pltpu.make_async_copy(rows_v.at[b], out_slice(0), ssem).wait()

        idx_load(0, 0)
        gather_start(0)

        def step(i, b, p):
            gather_wait(b)
            store_start(i, b)
            if b == 0:
                pl.when(i >= 1)(lambda: store_wait(1))
                idx_load(i + 1, 1)
                gather_start(1)
            else:
                store_wait(0)

                def launch_next():
                    idx_load(i + 1, 0)
                    gather_start(0)

                pl.when(p < n_pairs - 1)(launch_next)

        def pair_body(p, carry):
            step(2 * p, 0, p)
            step(2 * p + 1, 1, p)
            return carry

        lax.fori_loop(0, n_pairs, pair_body, 0)
        store_wait(1)

    return k


def kernel(words, weight):
    B = words.size
    V, D = weight.shape
    idx = words.reshape(B // _SUB, _SUB).astype(jnp.int32)
    out = _make_gather(B, V, D)(weight, idx)
    return out.reshape(*words.shape, D)


# speedup vs baseline: 1.1936x; 1.1908x over previous
import functools

import jax
import jax.numpy as jnp
from jax import lax
from jax.experimental import pallas as pl
from jax.experimental.pallas import tpu as pltpu
from jax.experimental.pallas import tpu_sc as plsc

_NC = 2
_NS = 16
_NW = _NC * _NS

_SUB = 128
_N_SUB = 8
_CHUNK = _SUB * _N_SUB


@functools.lru_cache(maxsize=None)
def _make_gather(B, V, D):
    b_per_w = B // _NW
    n_chunks = b_per_w // _CHUNK
    assert b_per_w * _NW == B and n_chunks * _CHUNK == b_per_w

    mesh = plsc.VectorSubcoreMesh(core_axis_name="c", subcore_axis_name="s")

    @functools.partial(
        pl.kernel,
        out_type=jax.ShapeDtypeStruct((B, D), jnp.float32),
        mesh=mesh,
        scratch_types=[
            pltpu.VMEM((_N_SUB, _SUB), jnp.int32),
            pltpu.VMEM((_CHUNK // 2, D), jnp.float32),
            pltpu.SemaphoreType.DMA,
        ],
        compiler_params=pltpu.CompilerParams(use_tc_tiling_on_sc=True),
    )
    def k(table_hbm, idx_hbm, out_hbm, idx_v, rows_v, sem):
        wid = lax.axis_index("s") * _NC + lax.axis_index("c")
        base = wid * b_per_w

        def body(i, carry):
            off = pl.multiple_of(base + i * _CHUNK, _CHUNK)
            row = pl.multiple_of(off // _SUB, _N_SUB)
            pltpu.sync_copy(idx_hbm.at[pl.ds(row, _N_SUB)], idx_v)
            for h in range(2):
                copies = [
                    pltpu.async_copy(
                        table_hbm.at[idx_v.at[h * (_N_SUB // 2) + j]],
                        rows_v.at[pl.ds(j * _SUB, _SUB)],
                        sem,
                    )
                    for j in range(_N_SUB // 2)
                ]
                for c in copies:
                    c.wait()
                pltpu.sync_copy(
                    rows_v, out_hbm.at[pl.ds(off + h * (_CHUNK // 2), _CHUNK // 2)]
                )
            return carry

        lax.fori_loop(0, n_chunks, body, 0)

    return k


def kernel(words, weight):
    B = words.size
    V, D = weight.shape
    wpad = jnp.pad(weight, ((0, 0), (0, 128 - D)))
    idx = words.reshape(B // _SUB, _SUB).astype(jnp.int32)
    out = _make_gather(B, V, 128)(wpad, idx)
    return out[:, :D].reshape(*words.shape, D)
